# baseline (speedup 1.0000x reference)
import functools

import jax
import jax.numpy as jnp
from jax import lax
from jax.experimental import pallas as pl
from jax.experimental.pallas import tpu as pltpu
import jax.experimental.pallas.tpu_sc as plsc

N = 10000
D = 128
G = 128
H = D // 2
NC = 2
NS = 16
CHUNK = 128
NPAD = 10112
ROWS_PER_TILE = NPAD // NS
DUMMY_ROW = N
NSLOT = 4


def _make_sc_agg(kpt):
  assert kpt % NSLOT == 0
  nq = kpt // NSLOT
  mesh = plsc.VectorSubcoreMesh(core_axis_name="c", subcore_axis_name="s")

  @functools.partial(
      pl.kernel,
      out_type=jax.ShapeDtypeStruct((NC, NPAD, H), jnp.float32),
      mesh=mesh,
      scratch_types=[
          pltpu.VMEM((kpt, CHUNK), jnp.int32),
          pltpu.VMEM((kpt, CHUNK), jnp.int32),
          pltpu.VMEM_SHARED((NPAD, H), jnp.float32),
      ] + [pltpu.VMEM((CHUNK, H), jnp.float32)] * NSLOT
        + [pltpu.SemaphoreType.DMA] * NSLOT,
      compiler_params=pltpu.CompilerParams(use_tc_tiling_on_sc=False),
  )
  def sc_agg(h2_hbm, srcs_hbm, dsts_hbm, zeros_hbm, out_hbm,
             src_v, dst_v, acc_sh, *rest):
    gbuf = rest[:NSLOT]
    gsem = rest[NSLOT:]
    c = lax.axis_index("c")
    s = lax.axis_index("s")
    pltpu.sync_copy(zeros_hbm,
                    acc_sh.at[pl.ds(s * ROWS_PER_TILE, ROWS_PER_TILE)])
    pltpu.sync_copy(srcs_hbm.at[s], src_v)
    pltpu.sync_copy(dsts_hbm.at[s], dst_v)
    plsc.subcore_barrier()
    hsrc = h2_hbm.at[c]

    def body(q, carry):
      for b in range(NSLOT):
        pltpu.async_copy(hsrc.at[src_v.at[q * NSLOT + b]], gbuf[b], gsem[b])
      for b in range(NSLOT):
        pltpu.make_async_copy(hsrc.at[src_v.at[0]], gbuf[b], gsem[b]).wait()
        pltpu.sync_copy(gbuf[b], acc_sh.at[dst_v.at[q * NSLOT + b]],
                        add=True)
      return carry

    lax.fori_loop(0, nq, body, 0)
    plsc.subcore_barrier()
    pltpu.sync_copy(acc_sh.at[pl.ds(s * ROWS_PER_TILE, ROWS_PER_TILE)],
                    out_hbm.at[c, pl.ds(s * ROWS_PER_TILE, ROWS_PER_TILE)])

  return sc_agg


def _layer_math(h, p0, p1, w1, b1, w2, b2, g, b):
  u = jnp.concatenate([h[:, :H] + p0, h[:, H:] + p1], axis=1)
  a = jnp.maximum(
      jnp.dot(u, w1, preferred_element_type=jnp.float32) + b1, 0.0)
  v = jnp.dot(a, w2, preferred_element_type=jnp.float32) + b2
  r = jnp.maximum(v, 0.0)
  mu = jnp.mean(r, axis=0, keepdims=True)
  var = jnp.mean(jnp.square(r - mu), axis=0, keepdims=True)
  return g * (r - mu) * lax.rsqrt(var + 1e-5) + b


def _tc_layer_body(h_ref, part_ref, w1_ref, b1_ref, w2_ref, b2_ref,
                   g_ref, b_ref, o_ref):
  h = jnp.concatenate([h_ref[0], h_ref[1]], axis=1)
  hn = _layer_math(h, part_ref[0, :N, :], part_ref[1, :N, :],
                   w1_ref[...], b1_ref[...], w2_ref[...], b2_ref[...],
                   g_ref[...], b_ref[...])
  o_ref[0] = hn[:, :H]
  o_ref[1] = hn[:, H:]


_tc_layer = pl.pallas_call(
    _tc_layer_body,
    out_shape=jax.ShapeDtypeStruct((NC, N, H), jnp.float32),
)


def _tc_last_body(h_ref, part_ref, w1_ref, b1_ref, w2_ref, b2_ref,
                  g_ref, b_ref, batch_ref, lw_ref, lb_ref, h_out, o_out):
  h = jnp.concatenate([h_ref[0], h_ref[1]], axis=1)
  hn = _layer_math(h, part_ref[0, :N, :], part_ref[1, :N, :],
                   w1_ref[...], b1_ref[...], w2_ref[...], b2_ref[...],
                   g_ref[...], b_ref[...])
  h_out[...] = hn
  ids = batch_ref[...]
  oh = (ids == lax.broadcasted_iota(jnp.int32, (N, G), 1)).astype(jnp.float32)
  xpool = lax.dot_general(oh, hn,
                          dimension_numbers=(((0,), (0,)), ((), ())),
                          preferred_element_type=jnp.float32)
  o_out[...] = jnp.dot(xpool, lw_ref[...],
                       preferred_element_type=jnp.float32) + lb_ref[...]


_tc_last = pl.pallas_call(
    _tc_last_body,
    out_shape=(jax.ShapeDtypeStruct((N, D), jnp.float32),
               jax.ShapeDtypeStruct((G, 2 * D), jnp.float32)),
)


@jax.jit
def kernel(x, edge_index, batch, c0_W1, c0_b1, c0_W2, c0_b2, c1_W1, c1_b1,
           c1_W2, c1_b2, c2_W1, c2_b1, c2_W2, c2_b2, bn0_g, bn0_b, bn1_g,
           bn1_b, bn2_g, bn2_b, lin0_W, lin0_b):
  src = edge_index[0].astype(jnp.int32)
  dst = edge_index[1].astype(jnp.int32)
  e = src.shape[0]
  kpt = -(-e // (NS * CHUNK * NSLOT)) * NSLOT
  e_pad = kpt * NS * CHUNK
  src_p = jnp.concatenate(
      [src, jnp.zeros((e_pad - e,), jnp.int32)]).reshape(NS, kpt, CHUNK)
  dst_p = jnp.concatenate(
      [dst, jnp.full((e_pad - e,), DUMMY_ROW, jnp.int32)]).reshape(
          NS, kpt, CHUNK)
  zeros = jnp.zeros((ROWS_PER_TILE, H), jnp.float32)
  sc_agg = _make_sc_agg(kpt)

  layers = [
      (c0_W1, c0_b1, c0_W2, c0_b2, bn0_g, bn0_b),
      (c1_W1, c1_b1, c1_W2, c1_b2, bn1_g, bn1_b),
      (c2_W1, c2_b1, c2_W2, c2_b2, bn2_g, bn2_b),
  ]
  h2 = jnp.stack([x[:, :H], x[:, H:]])
  for (w1, b1, w2, b2, g, b) in layers[:2]:
    part = sc_agg(h2, src_p, dst_p, zeros)
    h2 = _tc_layer(h2, part, w1, b1.reshape(1, D), w2, b2.reshape(1, D),
                   g.reshape(1, D), b.reshape(1, D))
  (w1, b1, w2, b2, g, b) = layers[2]
  part = sc_agg(h2, src_p, dst_p, zeros)
  h, out = _tc_last(h2, part, w1, b1.reshape(1, D), w2, b2.reshape(1, D),
                    g.reshape(1, D), b.reshape(1, D),
                    batch.astype(jnp.int32).reshape(N, 1), lin0_W,
                    lin0_b.reshape(1, 2 * D))
  return (out, h)


# speedup vs baseline: 1.0160x; 1.0160x over previous
import functools

import jax
import jax.numpy as jnp
from jax import lax
from jax.experimental import pallas as pl
from jax.experimental.pallas import tpu as pltpu
import jax.experimental.pallas.tpu_sc as plsc

N = 10000
D = 128
G = 128
H = D // 2
NC = 2
NS = 16
CHUNK = 128
NPAD = 10112
ROWS_PER_TILE = NPAD // NS
DUMMY_ROW = N
NSLOT = 4


def _make_sc_agg(kpt):
  assert kpt % NSLOT == 0
  nq = kpt // NSLOT
  mesh = plsc.VectorSubcoreMesh(core_axis_name="c", subcore_axis_name="s")

  @functools.partial(
      pl.kernel,
      out_type=jax.ShapeDtypeStruct((NC, NPAD, H), jnp.float32),
      mesh=mesh,
      scratch_types=[
          pltpu.VMEM((kpt, CHUNK), jnp.int32),
          pltpu.VMEM((kpt, CHUNK), jnp.int32),
          pltpu.VMEM_SHARED((NPAD, H), jnp.float32),
      ] + [pltpu.VMEM((CHUNK, H), jnp.float32)] * NSLOT
        + [pltpu.SemaphoreType.DMA] * (2 * NSLOT),
      compiler_params=pltpu.CompilerParams(use_tc_tiling_on_sc=False),
  )
  def sc_agg(h2_hbm, srcs_hbm, dsts_hbm, zeros_hbm, out_hbm,
             src_v, dst_v, acc_sh, *rest):
    gbuf = rest[:NSLOT]
    gsem = rest[NSLOT:2 * NSLOT]
    ssem = rest[2 * NSLOT:]
    c = lax.axis_index("c")
    s = lax.axis_index("s")
    pltpu.sync_copy(zeros_hbm,
                    acc_sh.at[pl.ds(s * ROWS_PER_TILE, ROWS_PER_TILE)])
    pltpu.sync_copy(srcs_hbm.at[s], src_v)
    pltpu.sync_copy(dsts_hbm.at[s], dst_v)
    plsc.subcore_barrier()
    hsrc = h2_hbm.at[c]

    def body(q, carry):
      @pl.when(q > 0)
      def _():
        for b in range(NSLOT):
          pltpu.make_async_copy(gbuf[b], acc_sh.at[dst_v.at[0]],
                                ssem[b]).wait()
      for b in range(NSLOT):
        pltpu.async_copy(hsrc.at[src_v.at[q * NSLOT + b]], gbuf[b], gsem[b])
      for b in range(NSLOT):
        pltpu.make_async_copy(hsrc.at[src_v.at[0]], gbuf[b], gsem[b]).wait()
        pltpu.async_copy(gbuf[b], acc_sh.at[dst_v.at[q * NSLOT + b]],
                         ssem[b], add=True)
      return carry

    lax.fori_loop(0, nq, body, 0)
    for b in range(NSLOT):
      pltpu.make_async_copy(gbuf[b], acc_sh.at[dst_v.at[0]], ssem[b]).wait()
    plsc.subcore_barrier()
    pltpu.sync_copy(acc_sh.at[pl.ds(s * ROWS_PER_TILE, ROWS_PER_TILE)],
                    out_hbm.at[c, pl.ds(s * ROWS_PER_TILE, ROWS_PER_TILE)])

  return sc_agg


def _layer_math(h, p0, p1, w1, b1, w2, b2, g, b):
  u = jnp.concatenate([h[:, :H] + p0, h[:, H:] + p1], axis=1)
  a = jnp.maximum(
      jnp.dot(u, w1, preferred_element_type=jnp.float32) + b1, 0.0)
  v = jnp.dot(a, w2, preferred_element_type=jnp.float32) + b2
  r = jnp.maximum(v, 0.0)
  mu = jnp.mean(r, axis=0, keepdims=True)
  var = jnp.mean(jnp.square(r - mu), axis=0, keepdims=True)
  return g * (r - mu) * lax.rsqrt(var + 1e-5) + b


def _tc_layer_body(h_ref, part_ref, w1_ref, b1_ref, w2_ref, b2_ref,
                   g_ref, b_ref, o_ref):
  h = jnp.concatenate([h_ref[0], h_ref[1]], axis=1)
  hn = _layer_math(h, part_ref[0, :N, :], part_ref[1, :N, :],
                   w1_ref[...], b1_ref[...], w2_ref[...], b2_ref[...],
                   g_ref[...], b_ref[...])
  o_ref[0] = hn[:, :H]
  o_ref[1] = hn[:, H:]


_tc_layer = pl.pallas_call(
    _tc_layer_body,
    out_shape=jax.ShapeDtypeStruct((NC, N, H), jnp.float32),
)


def _tc_last_body(h_ref, part_ref, w1_ref, b1_ref, w2_ref, b2_ref,
                  g_ref, b_ref, batch_ref, lw_ref, lb_ref, h_out, o_out):
  h = jnp.concatenate([h_ref[0], h_ref[1]], axis=1)
  hn = _layer_math(h, part_ref[0, :N, :], part_ref[1, :N, :],
                   w1_ref[...], b1_ref[...], w2_ref[...], b2_ref[...],
                   g_ref[...], b_ref[...])
  h_out[...] = hn
  ids = batch_ref[...]
  oh = (ids == lax.broadcasted_iota(jnp.int32, (N, G), 1)).astype(jnp.float32)
  xpool = lax.dot_general(oh, hn,
                          dimension_numbers=(((0,), (0,)), ((), ())),
                          preferred_element_type=jnp.float32)
  o_out[...] = jnp.dot(xpool, lw_ref[...],
                       preferred_element_type=jnp.float32) + lb_ref[...]


_tc_last = pl.pallas_call(
    _tc_last_body,
    out_shape=(jax.ShapeDtypeStruct((N, D), jnp.float32),
               jax.ShapeDtypeStruct((G, 2 * D), jnp.float32)),
)


@jax.jit
def kernel(x, edge_index, batch, c0_W1, c0_b1, c0_W2, c0_b2, c1_W1, c1_b1,
           c1_W2, c1_b2, c2_W1, c2_b1, c2_W2, c2_b2, bn0_g, bn0_b, bn1_g,
           bn1_b, bn2_g, bn2_b, lin0_W, lin0_b):
  src = edge_index[0].astype(jnp.int32)
  dst = edge_index[1].astype(jnp.int32)
  e = src.shape[0]
  kpt = -(-e // (NS * CHUNK * NSLOT)) * NSLOT
  e_pad = kpt * NS * CHUNK
  src_p = jnp.concatenate(
      [src, jnp.zeros((e_pad - e,), jnp.int32)]).reshape(NS, kpt, CHUNK)
  pad_dst = DUMMY_ROW + jnp.arange(e_pad - e, dtype=jnp.int32) % (NPAD - N)
  dst_p = jnp.concatenate([dst, pad_dst]).reshape(NS, kpt, CHUNK)
  zeros = jnp.zeros((ROWS_PER_TILE, H), jnp.float32)
  sc_agg = _make_sc_agg(kpt)

  layers = [
      (c0_W1, c0_b1, c0_W2, c0_b2, bn0_g, bn0_b),
      (c1_W1, c1_b1, c1_W2, c1_b2, bn1_g, bn1_b),
      (c2_W1, c2_b1, c2_W2, c2_b2, bn2_g, bn2_b),
  ]
  h2 = jnp.stack([x[:, :H], x[:, H:]])
  for (w1, b1, w2, b2, g, b) in layers[:2]:
    part = sc_agg(h2, src_p, dst_p, zeros)
    h2 = _tc_layer(h2, part, w1, b1.reshape(1, D), w2, b2.reshape(1, D),
                   g.reshape(1, D), b.reshape(1, D))
  (w1, b1, w2, b2, g, b) = layers[2]
  part = sc_agg(h2, src_p, dst_p, zeros)
  h, out = _tc_last(h2, part, w1, b1.reshape(1, D), w2, b2.reshape(1, D),
                    g.reshape(1, D), b.reshape(1, D),
                    batch.astype(jnp.int32).reshape(N, 1), lin0_W,
                    lin0_b.reshape(1, 2 * D))
  return (out, h)


# speedup vs baseline: 1.0433x; 1.0269x over previous
import functools

import jax
import jax.numpy as jnp
from jax import lax
from jax.experimental import pallas as pl
from jax.experimental.pallas import tpu as pltpu
import jax.experimental.pallas.tpu_sc as plsc

N = 10000
D = 128
G = 128
H = D // 2
NC = 2
NS = 16
CHUNK = 256

NPAD = 10112
ROWS_PER_TILE = NPAD // NS
DUMMY_ROW = N
NSLOT = 2


def _make_sc_agg(kpt):
  assert kpt % NSLOT == 0
  nq = kpt // NSLOT
  mesh = plsc.VectorSubcoreMesh(core_axis_name="c", subcore_axis_name="s")

  @functools.partial(
      pl.kernel,
      out_type=jax.ShapeDtypeStruct((NC, NPAD, H), jnp.float32),
      mesh=mesh,
      scratch_types=[
          pltpu.VMEM((kpt, CHUNK), jnp.int32),
          pltpu.VMEM((kpt, CHUNK), jnp.int32),
          pltpu.VMEM_SHARED((NPAD, H), jnp.float32),
      ] + [pltpu.VMEM((CHUNK, H), jnp.float32)] * NSLOT
        + [pltpu.SemaphoreType.DMA] * (2 * NSLOT),
      compiler_params=pltpu.CompilerParams(use_tc_tiling_on_sc=False),
  )
  def sc_agg(h2_hbm, srcs_hbm, dsts_hbm, zeros_hbm, out_hbm,
             src_v, dst_v, acc_sh, *rest):
    gbuf = rest[:NSLOT]
    gsem = rest[NSLOT:2 * NSLOT]
    ssem = rest[2 * NSLOT:]
    c = lax.axis_index("c")
    s = lax.axis_index("s")
    pltpu.sync_copy(zeros_hbm,
                    acc_sh.at[pl.ds(s * ROWS_PER_TILE, ROWS_PER_TILE)])
    pltpu.sync_copy(srcs_hbm.at[s], src_v)
    pltpu.sync_copy(dsts_hbm.at[s], dst_v)
    plsc.subcore_barrier()
    hsrc = h2_hbm.at[c]

    def body(q, carry):
      @pl.when(q > 0)
      def _():
        for b in range(NSLOT):
          pltpu.make_async_copy(gbuf[b], acc_sh.at[dst_v.at[0]],
                                ssem[b]).wait()
      for b in range(NSLOT):
        pltpu.async_copy(hsrc.at[src_v.at[q * NSLOT + b]], gbuf[b], gsem[b])
      for b in range(NSLOT):
        pltpu.make_async_copy(hsrc.at[src_v.at[0]], gbuf[b], gsem[b]).wait()
        pltpu.async_copy(gbuf[b], acc_sh.at[dst_v.at[q * NSLOT + b]],
                         ssem[b], add=True)
      return carry

    lax.fori_loop(0, nq, body, 0)
    for b in range(NSLOT):
      pltpu.make_async_copy(gbuf[b], acc_sh.at[dst_v.at[0]], ssem[b]).wait()
    plsc.subcore_barrier()
    pltpu.sync_copy(acc_sh.at[pl.ds(s * ROWS_PER_TILE, ROWS_PER_TILE)],
                    out_hbm.at[c, pl.ds(s * ROWS_PER_TILE, ROWS_PER_TILE)])

  return sc_agg


def _layer_math(h, p0, p1, w1, b1, w2, b2, g, b):
  u = jnp.concatenate([h[:, :H] + p0, h[:, H:] + p1], axis=1)
  a = jnp.maximum(
      jnp.dot(u, w1, preferred_element_type=jnp.float32) + b1, 0.0)
  v = jnp.dot(a, w2, preferred_element_type=jnp.float32) + b2
  r = jnp.maximum(v, 0.0)
  mu = jnp.mean(r, axis=0, keepdims=True)
  var = jnp.mean(jnp.square(r - mu), axis=0, keepdims=True)
  return g * (r - mu) * lax.rsqrt(var + 1e-5) + b


def _tc_layer_body(h_ref, part_ref, w1_ref, b1_ref, w2_ref, b2_ref,
                   g_ref, b_ref, o_ref):
  h = jnp.concatenate([h_ref[0], h_ref[1]], axis=1)
  hn = _layer_math(h, part_ref[0, :N, :], part_ref[1, :N, :],
                   w1_ref[...], b1_ref[...], w2_ref[...], b2_ref[...],
                   g_ref[...], b_ref[...])
  o_ref[0] = hn[:, :H]
  o_ref[1] = hn[:, H:]


_tc_layer = pl.pallas_call(
    _tc_layer_body,
    out_shape=jax.ShapeDtypeStruct((NC, N, H), jnp.float32),
)


def _tc_last_body(h_ref, part_ref, w1_ref, b1_ref, w2_ref, b2_ref,
                  g_ref, b_ref, batch_ref, lw_ref, lb_ref, h_out, o_out):
  h = jnp.concatenate([h_ref[0], h_ref[1]], axis=1)
  hn = _layer_math(h, part_ref[0, :N, :], part_ref[1, :N, :],
                   w1_ref[...], b1_ref[...], w2_ref[...], b2_ref[...],
                   g_ref[...], b_ref[...])
  h_out[...] = hn
  ids = batch_ref[...]
  oh = (ids == lax.broadcasted_iota(jnp.int32, (N, G), 1)).astype(jnp.float32)
  xpool = lax.dot_general(oh, hn,
                          dimension_numbers=(((0,), (0,)), ((), ())),
                          preferred_element_type=jnp.float32)
  o_out[...] = jnp.dot(xpool, lw_ref[...],
                       preferred_element_type=jnp.float32) + lb_ref[...]


_tc_last = pl.pallas_call(
    _tc_last_body,
    out_shape=(jax.ShapeDtypeStruct((N, D), jnp.float32),
               jax.ShapeDtypeStruct((G, 2 * D), jnp.float32)),
)


@jax.jit
def kernel(x, edge_index, batch, c0_W1, c0_b1, c0_W2, c0_b2, c1_W1, c1_b1,
           c1_W2, c1_b2, c2_W1, c2_b1, c2_W2, c2_b2, bn0_g, bn0_b, bn1_g,
           bn1_b, bn2_g, bn2_b, lin0_W, lin0_b):
  src = edge_index[0].astype(jnp.int32)
  dst = edge_index[1].astype(jnp.int32)
  e = src.shape[0]
  kpt = -(-e // (NS * CHUNK * NSLOT)) * NSLOT
  e_pad = kpt * NS * CHUNK
  src_p = jnp.concatenate(
      [src, jnp.zeros((e_pad - e,), jnp.int32)]).reshape(NS, kpt, CHUNK)
  pad_dst = DUMMY_ROW + jnp.arange(e_pad - e, dtype=jnp.int32) % (NPAD - N)
  dst_p = jnp.concatenate([dst, pad_dst]).reshape(NS, kpt, CHUNK)
  zeros = jnp.zeros((ROWS_PER_TILE, H), jnp.float32)
  sc_agg = _make_sc_agg(kpt)

  layers = [
      (c0_W1, c0_b1, c0_W2, c0_b2, bn0_g, bn0_b),
      (c1_W1, c1_b1, c1_W2, c1_b2, bn1_g, bn1_b),
      (c2_W1, c2_b1, c2_W2, c2_b2, bn2_g, bn2_b),
  ]
  h2 = jnp.stack([x[:, :H], x[:, H:]])
  for (w1, b1, w2, b2, g, b) in layers[:2]:
    part = sc_agg(h2, src_p, dst_p, zeros)
    h2 = _tc_layer(h2, part, w1, b1.reshape(1, D), w2, b2.reshape(1, D),
                   g.reshape(1, D), b.reshape(1, D))
  (w1, b1, w2, b2, g, b) = layers[2]
  part = sc_agg(h2, src_p, dst_p, zeros)
  h, out = _tc_last(h2, part, w1, b1.reshape(1, D), w2, b2.reshape(1, D),
                    g.reshape(1, D), b.reshape(1, D),
                    batch.astype(jnp.int32).reshape(N, 1), lin0_W,
                    lin0_b.reshape(1, 2 * D))
  return (out, h)


# speedup vs baseline: 1.5670x; 1.5019x over previous
import functools

import jax
import jax.numpy as jnp
from jax import lax
from jax.experimental import pallas as pl
from jax.experimental.pallas import tpu as pltpu
import jax.experimental.pallas.tpu_sc as plsc

N = 10000
D = 128
G = 128
H = D // 2
NC = 2
NS = 16
CHUNK = 64

NPAD = 10112
ROWS_PER_TILE = NPAD // NS
DUMMY_ROW = N
NSLOT = 2


def _make_sc_agg(kpt):
  assert kpt % NSLOT == 0
  nq = kpt // NSLOT
  mesh = plsc.VectorSubcoreMesh(core_axis_name="c", subcore_axis_name="s")

  @functools.partial(
      pl.kernel,
      out_type=jax.ShapeDtypeStruct((NC, NPAD, H), jnp.float32),
      mesh=mesh,
      scratch_types=[
          pltpu.VMEM((kpt, CHUNK), jnp.int32),
          pltpu.VMEM((kpt, CHUNK), jnp.int32),
          pltpu.VMEM_SHARED((NPAD, H), jnp.float32),
          pltpu.VMEM_SHARED((N, H), jnp.float32),
      ] + [pltpu.VMEM((CHUNK, H), jnp.float32)] * NSLOT
        + [pltpu.SemaphoreType.DMA] * (2 * NSLOT),
      compiler_params=pltpu.CompilerParams(use_tc_tiling_on_sc=False),
  )
  def sc_agg(h2_hbm, srcs_hbm, dsts_hbm, zeros_hbm, out_hbm,
             src_v, dst_v, acc_sh, h_sh, *rest):
    gbuf = rest[:NSLOT]
    gsem = rest[NSLOT:2 * NSLOT]
    ssem = rest[2 * NSLOT:]
    c = lax.axis_index("c")
    s = lax.axis_index("s")
    pltpu.sync_copy(zeros_hbm,
                    acc_sh.at[pl.ds(s * ROWS_PER_TILE, ROWS_PER_TILE)])
    pltpu.sync_copy(h2_hbm.at[c, pl.ds(s * (N // NS), N // NS)],
                    h_sh.at[pl.ds(s * (N // NS), N // NS)])
    pltpu.sync_copy(srcs_hbm.at[s], src_v)
    pltpu.sync_copy(dsts_hbm.at[s], dst_v)
    plsc.subcore_barrier()

    def body(q, carry):
      @pl.when(q > 0)
      def _():
        for b in range(NSLOT):
          pltpu.make_async_copy(gbuf[b], acc_sh.at[dst_v.at[0]],
                                ssem[b]).wait()
      for b in range(NSLOT):
        pltpu.async_copy(h_sh.at[src_v.at[q * NSLOT + b]], gbuf[b], gsem[b])
      for b in range(NSLOT):
        pltpu.make_async_copy(h_sh.at[src_v.at[0]], gbuf[b], gsem[b]).wait()
        pltpu.async_copy(gbuf[b], acc_sh.at[dst_v.at[q * NSLOT + b]],
                         ssem[b], add=True)
      return carry

    lax.fori_loop(0, nq, body, 0)
    for b in range(NSLOT):
      pltpu.make_async_copy(gbuf[b], acc_sh.at[dst_v.at[0]], ssem[b]).wait()
    plsc.subcore_barrier()
    pltpu.sync_copy(acc_sh.at[pl.ds(s * ROWS_PER_TILE, ROWS_PER_TILE)],
                    out_hbm.at[c, pl.ds(s * ROWS_PER_TILE, ROWS_PER_TILE)])

  return sc_agg


def _layer_math(h, p0, p1, w1, b1, w2, b2, g, b):
  u = jnp.concatenate([h[:, :H] + p0, h[:, H:] + p1], axis=1)
  a = jnp.maximum(
      jnp.dot(u, w1, preferred_element_type=jnp.float32) + b1, 0.0)
  v = jnp.dot(a, w2, preferred_element_type=jnp.float32) + b2
  r = jnp.maximum(v, 0.0)
  mu = jnp.mean(r, axis=0, keepdims=True)
  var = jnp.mean(jnp.square(r - mu), axis=0, keepdims=True)
  return g * (r - mu) * lax.rsqrt(var + 1e-5) + b


def _tc_layer_body(h_ref, part_ref, w1_ref, b1_ref, w2_ref, b2_ref,
                   g_ref, b_ref, o_ref):
  h = jnp.concatenate([h_ref[0], h_ref[1]], axis=1)
  hn = _layer_math(h, part_ref[0, :N, :], part_ref[1, :N, :],
                   w1_ref[...], b1_ref[...], w2_ref[...], b2_ref[...],
                   g_ref[...], b_ref[...])
  o_ref[0] = hn[:, :H]
  o_ref[1] = hn[:, H:]


_tc_layer = pl.pallas_call(
    _tc_layer_body,
    out_shape=jax.ShapeDtypeStruct((NC, N, H), jnp.float32),
)


def _tc_last_body(h_ref, part_ref, w1_ref, b1_ref, w2_ref, b2_ref,
                  g_ref, b_ref, batch_ref, lw_ref, lb_ref, h_out, o_out):
  h = jnp.concatenate([h_ref[0], h_ref[1]], axis=1)
  hn = _layer_math(h, part_ref[0, :N, :], part_ref[1, :N, :],
                   w1_ref[...], b1_ref[...], w2_ref[...], b2_ref[...],
                   g_ref[...], b_ref[...])
  h_out[...] = hn
  ids = batch_ref[...]
  oh = (ids == lax.broadcasted_iota(jnp.int32, (N, G), 1)).astype(jnp.float32)
  xpool = lax.dot_general(oh, hn,
                          dimension_numbers=(((0,), (0,)), ((), ())),
                          preferred_element_type=jnp.float32)
  o_out[...] = jnp.dot(xpool, lw_ref[...],
                       preferred_element_type=jnp.float32) + lb_ref[...]


_tc_last = pl.pallas_call(
    _tc_last_body,
    out_shape=(jax.ShapeDtypeStruct((N, D), jnp.float32),
               jax.ShapeDtypeStruct((G, 2 * D), jnp.float32)),
)


@jax.jit
def kernel(x, edge_index, batch, c0_W1, c0_b1, c0_W2, c0_b2, c1_W1, c1_b1,
           c1_W2, c1_b2, c2_W1, c2_b1, c2_W2, c2_b2, bn0_g, bn0_b, bn1_g,
           bn1_b, bn2_g, bn2_b, lin0_W, lin0_b):
  src = edge_index[0].astype(jnp.int32)
  dst = edge_index[1].astype(jnp.int32)
  e = src.shape[0]
  kpt = -(-e // (NS * CHUNK * NSLOT)) * NSLOT
  e_pad = kpt * NS * CHUNK
  src_p = jnp.concatenate(
      [src, jnp.zeros((e_pad - e,), jnp.int32)]).reshape(NS, kpt, CHUNK)
  pad_dst = DUMMY_ROW + jnp.arange(e_pad - e, dtype=jnp.int32) % (NPAD - N)
  dst_p = jnp.concatenate([dst, pad_dst]).reshape(NS, kpt, CHUNK)
  zeros = jnp.zeros((ROWS_PER_TILE, H), jnp.float32)
  sc_agg = _make_sc_agg(kpt)

  layers = [
      (c0_W1, c0_b1, c0_W2, c0_b2, bn0_g, bn0_b),
      (c1_W1, c1_b1, c1_W2, c1_b2, bn1_g, bn1_b),
      (c2_W1, c2_b1, c2_W2, c2_b2, bn2_g, bn2_b),
  ]
  h2 = jnp.stack([x[:, :H], x[:, H:]])
  for (w1, b1, w2, b2, g, b) in layers[:2]:
    part = sc_agg(h2, src_p, dst_p, zeros)
    h2 = _tc_layer(h2, part, w1, b1.reshape(1, D), w2, b2.reshape(1, D),
                   g.reshape(1, D), b.reshape(1, D))
  (w1, b1, w2, b2, g, b) = layers[2]
  part = sc_agg(h2, src_p, dst_p, zeros)
  h, out = _tc_last(h2, part, w1, b1.reshape(1, D), w2, b2.reshape(1, D),
                    g.reshape(1, D), b.reshape(1, D),
                    batch.astype(jnp.int32).reshape(N, 1), lin0_W,
                    lin0_b.reshape(1, 2 * D))
  return (out, h)


# speedup vs baseline: 2.1735x; 1.3871x over previous
import functools

import jax
import jax.numpy as jnp
from jax import lax
from jax.experimental import pallas as pl
from jax.experimental.pallas import tpu as pltpu
import jax.experimental.pallas.tpu_sc as plsc

N = 10000
D = 128
G = 128
H = D // 2
NC = 2
NS = 16
CHUNK = 32

NPAD = 10112
ROWS_PER_TILE = NPAD // NS
DUMMY_ROW = N
NSLOT = 8


def _make_sc_agg(kpt):
  assert kpt % NSLOT == 0
  nq = kpt // NSLOT
  mesh = plsc.VectorSubcoreMesh(core_axis_name="c", subcore_axis_name="s")

  @functools.partial(
      pl.kernel,
      out_type=jax.ShapeDtypeStruct((NC, NPAD, H), jnp.float32),
      mesh=mesh,
      scratch_types=[
          pltpu.VMEM((2 * NSLOT, CHUNK), jnp.int32),
          pltpu.VMEM((2 * NSLOT, CHUNK), jnp.int32),
          pltpu.VMEM_SHARED((NPAD, H), jnp.float32),
          pltpu.VMEM_SHARED((N, H), jnp.float32),
      ] + [pltpu.VMEM((CHUNK, H), jnp.float32)] * NSLOT
        + [pltpu.SemaphoreType.DMA] * (2 * NSLOT + 1),
      compiler_params=pltpu.CompilerParams(use_tc_tiling_on_sc=False),
  )
  def sc_agg(h2_hbm, srcs_hbm, dsts_hbm, zeros_hbm, out_hbm,
             src_v, dst_v, acc_sh, h_sh, *rest):
    gbuf = rest[:NSLOT]
    gsem = rest[NSLOT:2 * NSLOT]
    ssem = rest[2 * NSLOT:3 * NSLOT]
    isem = rest[3 * NSLOT]
    c = lax.axis_index("c")
    s = lax.axis_index("s")
    pltpu.sync_copy(zeros_hbm,
                    acc_sh.at[pl.ds(s * ROWS_PER_TILE, ROWS_PER_TILE)])
    pltpu.sync_copy(h2_hbm.at[c, pl.ds(s * (N // NS), N // NS)],
                    h_sh.at[pl.ds(s * (N // NS), N // NS)])
    pltpu.async_copy(srcs_hbm.at[s, pl.ds(0, NSLOT)],
                     src_v.at[pl.ds(0, NSLOT)], isem)
    pltpu.async_copy(dsts_hbm.at[s, pl.ds(0, NSLOT)],
                     dst_v.at[pl.ds(0, NSLOT)], isem)
    plsc.subcore_barrier()

    def body(q, carry):
      p = lax.rem(q, 2)
      mybase = p * NSLOT
      nxbase = NSLOT - mybase
      @pl.when(q > 0)
      def _():
        for b in range(NSLOT):
          pltpu.make_async_copy(gbuf[b], acc_sh.at[dst_v.at[0]],
                                ssem[b]).wait()
      pltpu.make_async_copy(srcs_hbm.at[s, pl.ds(0, NSLOT)],
                            src_v.at[pl.ds(0, NSLOT)], isem).wait()
      pltpu.make_async_copy(srcs_hbm.at[s, pl.ds(0, NSLOT)],
                            src_v.at[pl.ds(0, NSLOT)], isem).wait()
      pltpu.async_copy(srcs_hbm.at[s, pl.ds((q + 1) * NSLOT, NSLOT)],
                       src_v.at[pl.ds(nxbase, NSLOT)], isem)
      pltpu.async_copy(dsts_hbm.at[s, pl.ds((q + 1) * NSLOT, NSLOT)],
                       dst_v.at[pl.ds(nxbase, NSLOT)], isem)
      for b in range(NSLOT):
        pltpu.async_copy(h_sh.at[src_v.at[mybase + b]], gbuf[b], gsem[b])
      for b in range(NSLOT):
        pltpu.make_async_copy(h_sh.at[src_v.at[0]], gbuf[b], gsem[b]).wait()
        pltpu.async_copy(gbuf[b], acc_sh.at[dst_v.at[mybase + b]],
                         ssem[b], add=True)
      return carry

    lax.fori_loop(0, nq, body, 0)
    for b in range(NSLOT):
      pltpu.make_async_copy(gbuf[b], acc_sh.at[dst_v.at[0]], ssem[b]).wait()
    pltpu.make_async_copy(srcs_hbm.at[s, pl.ds(0, NSLOT)],
                          src_v.at[pl.ds(0, NSLOT)], isem).wait()
    pltpu.make_async_copy(srcs_hbm.at[s, pl.ds(0, NSLOT)],
                          src_v.at[pl.ds(0, NSLOT)], isem).wait()
    plsc.subcore_barrier()
    pltpu.sync_copy(acc_sh.at[pl.ds(s * ROWS_PER_TILE, ROWS_PER_TILE)],
                    out_hbm.at[c, pl.ds(s * ROWS_PER_TILE, ROWS_PER_TILE)])

  return sc_agg


def _layer_math(h, p0, p1, w1, b1, w2, b2, g, b):
  u = jnp.concatenate([h[:, :H] + p0, h[:, H:] + p1], axis=1)
  a = jnp.maximum(
      jnp.dot(u, w1, preferred_element_type=jnp.float32) + b1, 0.0)
  v = jnp.dot(a, w2, preferred_element_type=jnp.float32) + b2
  r = jnp.maximum(v, 0.0)
  mu = jnp.mean(r, axis=0, keepdims=True)
  var = jnp.mean(jnp.square(r - mu), axis=0, keepdims=True)
  return g * (r - mu) * lax.rsqrt(var + 1e-5) + b


def _tc_layer_body(h_ref, part_ref, w1_ref, b1_ref, w2_ref, b2_ref,
                   g_ref, b_ref, o_ref):
  h = jnp.concatenate([h_ref[0], h_ref[1]], axis=1)
  hn = _layer_math(h, part_ref[0, :N, :], part_ref[1, :N, :],
                   w1_ref[...], b1_ref[...], w2_ref[...], b2_ref[...],
                   g_ref[...], b_ref[...])
  o_ref[0] = hn[:, :H]
  o_ref[1] = hn[:, H:]


_tc_layer = pl.pallas_call(
    _tc_layer_body,
    out_shape=jax.ShapeDtypeStruct((NC, N, H), jnp.float32),
)


def _tc_last_body(h_ref, part_ref, w1_ref, b1_ref, w2_ref, b2_ref,
                  g_ref, b_ref, batch_ref, lw_ref, lb_ref, h_out, o_out):
  h = jnp.concatenate([h_ref[0], h_ref[1]], axis=1)
  hn = _layer_math(h, part_ref[0, :N, :], part_ref[1, :N, :],
                   w1_ref[...], b1_ref[...], w2_ref[...], b2_ref[...],
                   g_ref[...], b_ref[...])
  h_out[...] = hn
  ids = batch_ref[...]
  oh = (ids == lax.broadcasted_iota(jnp.int32, (N, G), 1)).astype(jnp.float32)
  xpool = lax.dot_general(oh, hn,
                          dimension_numbers=(((0,), (0,)), ((), ())),
                          preferred_element_type=jnp.float32)
  o_out[...] = jnp.dot(xpool, lw_ref[...],
                       preferred_element_type=jnp.float32) + lb_ref[...]


_tc_last = pl.pallas_call(
    _tc_last_body,
    out_shape=(jax.ShapeDtypeStruct((N, D), jnp.float32),
               jax.ShapeDtypeStruct((G, 2 * D), jnp.float32)),
)


@jax.jit
def kernel(x, edge_index, batch, c0_W1, c0_b1, c0_W2, c0_b2, c1_W1, c1_b1,
           c1_W2, c1_b2, c2_W1, c2_b1, c2_W2, c2_b2, bn0_g, bn0_b, bn1_g,
           bn1_b, bn2_g, bn2_b, lin0_W, lin0_b):
  src = edge_index[0].astype(jnp.int32)
  dst = edge_index[1].astype(jnp.int32)
  e = src.shape[0]
  kpt = -(-e // (NS * CHUNK * NSLOT)) * NSLOT
  e_pad = kpt * NS * CHUNK
  src_p = jnp.concatenate(
      [src, jnp.zeros((e_pad - e,), jnp.int32)]).reshape(NS, kpt, CHUNK)
  pad_dst = DUMMY_ROW + jnp.arange(e_pad - e, dtype=jnp.int32) % (NPAD - N)
  dst_p = jnp.concatenate([dst, pad_dst]).reshape(NS, kpt, CHUNK)
  src_p = jnp.concatenate(
      [src_p, jnp.zeros((NS, NSLOT, CHUNK), jnp.int32)], axis=1)
  dst_p = jnp.concatenate(
      [dst_p, jnp.full((NS, NSLOT, CHUNK), DUMMY_ROW, jnp.int32)], axis=1)
  zeros = jnp.zeros((ROWS_PER_TILE, H), jnp.float32)
  sc_agg = _make_sc_agg(kpt)

  layers = [
      (c0_W1, c0_b1, c0_W2, c0_b2, bn0_g, bn0_b),
      (c1_W1, c1_b1, c1_W2, c1_b2, bn1_g, bn1_b),
      (c2_W1, c2_b1, c2_W2, c2_b2, bn2_g, bn2_b),
  ]
  h2 = jnp.stack([x[:, :H], x[:, H:]])
  for (w1, b1, w2, b2, g, b) in layers[:2]:
    part = sc_agg(h2, src_p, dst_p, zeros)
    h2 = _tc_layer(h2, part, w1, b1.reshape(1, D), w2, b2.reshape(1, D),
                   g.reshape(1, D), b.reshape(1, D))
  (w1, b1, w2, b2, g, b) = layers[2]
  part = sc_agg(h2, src_p, dst_p, zeros)
  h, out = _tc_last(h2, part, w1, b1.reshape(1, D), w2, b2.reshape(1, D),
                    g.reshape(1, D), b.reshape(1, D),
                    batch.astype(jnp.int32).reshape(N, 1), lin0_W,
                    lin0_b.reshape(1, 2 * D))
  return (out, h)
